# baseline (speedup 1.0000x reference)
import functools

import jax
import jax.numpy as jnp
from jax import lax
from jax.experimental import pallas as pl
from jax.experimental.pallas import tpu as pltpu
from jax.experimental.pallas import tpu_sc as plsc

N_F = 32
B = 16384

_info = plsc.get_sparse_core_info()
NC, NS, L = _info.num_cores, _info.num_subcores, _info.num_lanes
NW = NC * NS
BPW = B // NW
CHUNK = 128
NCHUNK = BPW // CHUNK
NBUF = 12
NGRP = BPW // L
BLK = 128

_mesh = plsc.VectorSubcoreMesh(core_axis_name="c", subcore_axis_name="s")


@functools.partial(
    pl.kernel,
    out_type=jax.ShapeDtypeStruct((B,), jnp.float32),
    mesh=_mesh,
    compiler_params=pltpu.CompilerParams(needs_layout_passes=False,
                                         use_tc_tiling_on_sc=True),
    scratch_types=[
        pltpu.VMEM((NCHUNK, CHUNK), jnp.int32),
        pltpu.VMEM((NCHUNK, CHUNK), jnp.int32),
        pltpu.VMEM((NBUF, N_F, BLK), jnp.float32),
        pltpu.VMEM((NBUF, N_F, BLK), jnp.float32),
        pltpu.VMEM((BPW,), jnp.float32),
        pltpu.VMEM((BPW,), jnp.float32),
        pltpu.VMEM((BPW,), jnp.float32),
        pltpu.SemaphoreType.DMA,
    ] + [pltpu.SemaphoreType.DMA] * NBUF,
)
def _mf_kernel(users_hbm, items_hbm, uft_hbm, itft_hbm, ub_hbm, ib_hbm,
               out_hbm, idx_u, idx_i, bufu, bufi,
               ub_v, ib_v, out_v, bsem, *rsems):
    wid = lax.axis_index("s") * NC + lax.axis_index("c")
    base = wid * BPW

    for j in range(NCHUNK):
        src = pl.ds(base + j * CHUNK, CHUNK)
        pltpu.sync_copy(users_hbm.at[src], idx_u.at[j])
        pltpu.sync_copy(items_hbm.at[src], idx_i.at[j])

    bias_copies = []
    for j in range(NCHUNK):
        dst = pl.ds(j * CHUNK, CHUNK)
        bias_copies.append(pltpu.async_copy(ub_hbm.at[idx_u.at[j]], ub_v.at[dst], bsem))
        bias_copies.append(pltpu.async_copy(ib_hbm.at[idx_i.at[j]], ib_v.at[dst], bsem))

    lanes = lax.iota(jnp.int32, L)

    def sca(ref, e):
        v = ref[e >> 7, pl.ds(((e >> 4) & (CHUNK // L - 1)) * L, L)]
        return jnp.sum(jnp.where(lanes == (e & (L - 1)), v, 0))

    def fire(e, b):
        cu = pl.multiple_of((sca(idx_u, e) >> 7) * BLK, BLK)
        ci = pl.multiple_of((sca(idx_i, e) >> 7) * BLK, BLK)
        pltpu.async_copy(uft_hbm.at[:, pl.ds(cu, BLK)], bufu.at[b], rsems[b])
        pltpu.async_copy(itft_hbm.at[:, pl.ds(ci, BLK)], bufi.at[b], rsems[b])

    for b in range(NBUF):
        fire(b, b)

    for c in bias_copies:
        c.wait()

    def group(g, carry):
        acc = jnp.zeros((L,), jnp.float32)
        for r in range(L):
            e = g * L + r
            b = r % NBUF
            pltpu.make_async_copy(uft_hbm.at[:, pl.ds(0, BLK)], bufu.at[b],
                                  rsems[b]).wait()
            pltpu.make_async_copy(itft_hbm.at[:, pl.ds(0, BLK)], bufi.at[b],
                                  rsems[b]).wait()
            ju = jnp.full((L,), 0, jnp.int32) + (sca(idx_u, e) & (BLK - 1))
            ji = jnp.full((L,), 0, jnp.int32) + (sca(idx_i, e) & (BLK - 1))
            bv = jnp.full((L,), b, jnp.int32)
            u0 = plsc.load_gather(bufu, [bv, lanes, ju])
            u1 = plsc.load_gather(bufu, [bv, lanes + L, ju])
            t0 = plsc.load_gather(bufi, [bv, lanes, ji])
            t1 = plsc.load_gather(bufi, [bv, lanes + L, ji])
            s = jnp.sum(u0 * t0 + u1 * t1)
            acc = acc + s * (lanes == r).astype(jnp.float32)
            @pl.when(e + NBUF < BPW)
            def _():
                fire(e + NBUF, b)
        gs = pl.ds(g * L, L)
        x = acc + ub_v[gs] + ib_v[gs]
        out_v[gs] = 1.0 / (1.0 + jnp.exp(-x))
        return carry

    lax.fori_loop(0, NGRP, group, 0)

    pltpu.sync_copy(out_v, out_hbm.at[pl.ds(base, BPW)])


def kernel(users, items, user_factors, item_factors, user_biases, item_biases):
    users = users.astype(jnp.int32)
    items = items.astype(jnp.int32)
    return _mf_kernel(users, items, user_factors.T, item_factors.T,
                      user_biases.reshape(-1), item_biases.reshape(-1))


# speedup vs baseline: 1.0434x; 1.0434x over previous
import functools

import jax
import jax.numpy as jnp
from jax import lax
from jax.experimental import pallas as pl
from jax.experimental.pallas import tpu as pltpu
from jax.experimental.pallas import tpu_sc as plsc

N_F = 32
B = 16384

_info = plsc.get_sparse_core_info()
NC, NS, L = _info.num_cores, _info.num_subcores, _info.num_lanes
NW = NC * NS
BPW = B // NW
CHUNK = 128
NCHUNK = BPW // CHUNK
NBUF = 8
NGRP = BPW // L
BLK = 128

_mesh = plsc.VectorSubcoreMesh(core_axis_name="c", subcore_axis_name="s")


@functools.partial(
    pl.kernel,
    out_type=jax.ShapeDtypeStruct((B,), jnp.float32),
    mesh=_mesh,
    compiler_params=pltpu.CompilerParams(needs_layout_passes=False,
                                         use_tc_tiling_on_sc=True),
    scratch_types=[
        pltpu.VMEM((NCHUNK, CHUNK), jnp.int32),
        pltpu.VMEM((NCHUNK, CHUNK), jnp.int32),
        pltpu.VMEM((NBUF, N_F, BLK), jnp.float32),
        pltpu.VMEM((NBUF, N_F, BLK), jnp.float32),
        pltpu.VMEM((BPW,), jnp.float32),
        pltpu.VMEM((BPW,), jnp.float32),
        pltpu.VMEM((BPW,), jnp.float32),
        pltpu.SemaphoreType.DMA,
    ] + [pltpu.SemaphoreType.DMA] * NBUF,
)
def _mf_kernel(users_hbm, items_hbm, uft_hbm, itft_hbm, ub_hbm, ib_hbm,
               out_hbm, idx_u, idx_i, bufu, bufi,
               ub_v, ib_v, out_v, bsem, *rsems):
    wid = lax.axis_index("s") * NC + lax.axis_index("c")
    base = wid * BPW

    for j in range(NCHUNK):
        src = pl.ds(base + j * CHUNK, CHUNK)
        pltpu.sync_copy(users_hbm.at[src], idx_u.at[j])
        pltpu.sync_copy(items_hbm.at[src], idx_i.at[j])

    bias_copies = []
    for j in range(NCHUNK):
        dst = pl.ds(j * CHUNK, CHUNK)
        bias_copies.append(pltpu.async_copy(ub_hbm.at[idx_u.at[j]], ub_v.at[dst], bsem))
        bias_copies.append(pltpu.async_copy(ib_hbm.at[idx_i.at[j]], ib_v.at[dst], bsem))

    lanes = lax.iota(jnp.int32, L)

    def sca(ref, e):
        v = ref[e >> 7, pl.ds(((e >> 4) & (CHUNK // L - 1)) * L, L)]
        return jnp.sum(jnp.where(lanes == (e & (L - 1)), v, 0))

    def fire(e, b):
        cu = pl.multiple_of((sca(idx_u, e) >> 7) * BLK, BLK)
        ci = pl.multiple_of((sca(idx_i, e) >> 7) * BLK, BLK)
        pltpu.async_copy(uft_hbm.at[:, pl.ds(cu, BLK)], bufu.at[b], rsems[b])
        pltpu.async_copy(itft_hbm.at[:, pl.ds(ci, BLK)], bufi.at[b], rsems[b])

    for b in range(NBUF):
        fire(b, b)

    for c in bias_copies:
        c.wait()

    def group(g, carry):
        acc = jnp.zeros((L,), jnp.float32)
        for r in range(L):
            e = g * L + r
            b = r % NBUF
            pltpu.make_async_copy(uft_hbm.at[:, pl.ds(0, BLK)], bufu.at[b],
                                  rsems[b]).wait()
            pltpu.make_async_copy(itft_hbm.at[:, pl.ds(0, BLK)], bufi.at[b],
                                  rsems[b]).wait()
            ju = jnp.full((L,), 0, jnp.int32) + (sca(idx_u, e) & (BLK - 1))
            ji = jnp.full((L,), 0, jnp.int32) + (sca(idx_i, e) & (BLK - 1))
            bv = jnp.full((L,), b, jnp.int32)
            u0 = plsc.load_gather(bufu, [bv, lanes, ju])
            u1 = plsc.load_gather(bufu, [bv, lanes + L, ju])
            t0 = plsc.load_gather(bufi, [bv, lanes, ji])
            t1 = plsc.load_gather(bufi, [bv, lanes + L, ji])
            s = jnp.sum(u0 * t0 + u1 * t1)
            acc = acc + s * (lanes == r).astype(jnp.float32)
            @pl.when(e + NBUF < BPW)
            def _():
                fire(e + NBUF, b)
        gs = pl.ds(g * L, L)
        x = acc + ub_v[gs] + ib_v[gs]
        out_v[gs] = 1.0 / (1.0 + jnp.exp(-x))
        return carry

    lax.fori_loop(0, NGRP, group, 0)

    pltpu.sync_copy(out_v, out_hbm.at[pl.ds(base, BPW)])


def kernel(users, items, user_factors, item_factors, user_biases, item_biases):
    users = users.astype(jnp.int32)
    items = items.astype(jnp.int32)
    return _mf_kernel(users, items, user_factors.T, item_factors.T,
                      user_biases.reshape(-1), item_biases.reshape(-1))
